# baseline (speedup 1.0000x reference)
import functools

import jax
import jax.numpy as jnp
from jax.experimental import pallas as pl
from jax.experimental.pallas import tpu as pltpu
from jax.experimental.shard_map import shard_map
from jax.sharding import Mesh, PartitionSpec as P


def _support_kernel(v_ref, w_ref, o_ref):
    o_ref[...] = jnp.dot(v_ref[...], w_ref[...],
                         preferred_element_type=jnp.float32)


def _gc_kernel(ad_ref, aa_ref, s_ref, b_ref, o_ref):
    a = ad_ref[...] + aa_ref[...]
    o_ref[...] = (jnp.dot(a, s_ref[...], preferred_element_type=jnp.float32)
                  + b_ref[...])


def _impl(vertex, adj_distance, adj_angle, weights, bias2):
    n, in_f = vertex.shape
    out_f = weights.shape[1]
    n_loc = adj_distance.shape[0]

    support = pl.pallas_call(
        _support_kernel,
        out_shape=jax.ShapeDtypeStruct((n, out_f), jnp.float32),
    )(vertex, weights)

    tm = 40
    grid = (n_loc // tm,)

    return pl.pallas_call(
        _gc_kernel,
        grid=grid,
        in_specs=[
            pl.BlockSpec((tm, n), lambda m: (m, 0)),
            pl.BlockSpec((tm, n), lambda m: (m, 0)),
            pl.BlockSpec((n, out_f), lambda m: (0, 0)),
            pl.BlockSpec((1, out_f), lambda m: (0, 0)),
        ],
        out_specs=pl.BlockSpec((tm, out_f), lambda m: (m, 0)),
        out_shape=jax.ShapeDtypeStruct((n_loc, out_f), jnp.float32),
        compiler_params=pltpu.CompilerParams(
            dimension_semantics=("arbitrary",),
        ),
    )(adj_distance, adj_angle, support, bias2)


def kernel(vertex, adj_distance, adj_angle, weights, bias):
    n = vertex.shape[0]
    out_f = weights.shape[1]
    bias2 = bias.reshape(1, out_f)

    devs = jax.devices()
    ndev = len(devs)
    while ndev > 1 and (n % ndev != 0 or (n // ndev) % 40 != 0):
        ndev -= 1
    mesh = Mesh(devs[:ndev], ("x",))
    f = shard_map(
        _impl,
        mesh=mesh,
        in_specs=(P(), P("x", None), P("x", None), P(), P()),
        out_specs=P("x", None),
        check_rep=False,
    )
    return f(vertex, adj_distance, adj_angle, weights, bias2)


# speedup vs baseline: 3.5958x; 3.5958x over previous
import jax
import jax.numpy as jnp
from jax.experimental import pallas as pl
from jax.experimental.pallas import tpu as pltpu


def _gc_kernel(v_ref, w_ref, b_ref, ad_ref, aa_ref, o_ref, s_ref):
    @pl.when(pl.program_id(0) == 0)
    def _():
        s_ref[...] = jnp.dot(v_ref[...], w_ref[...],
                             preferred_element_type=jnp.float32)

    a = ad_ref[...] + aa_ref[...]
    o_ref[...] = (jnp.dot(a, s_ref[...], preferred_element_type=jnp.float32)
                  + b_ref[...])


def kernel(vertex, adj_distance, adj_angle, weights, bias):
    n, in_f = vertex.shape
    out_f = weights.shape[1]
    bias2 = bias.reshape(1, out_f)

    tm = 80
    grid = (n // tm,)

    return pl.pallas_call(
        _gc_kernel,
        grid=grid,
        in_specs=[
            pl.BlockSpec((n, in_f), lambda m: (0, 0)),
            pl.BlockSpec((in_f, out_f), lambda m: (0, 0)),
            pl.BlockSpec((1, out_f), lambda m: (0, 0)),
            pl.BlockSpec((tm, n), lambda m: (m, 0)),
            pl.BlockSpec((tm, n), lambda m: (m, 0)),
        ],
        out_specs=pl.BlockSpec((tm, out_f), lambda m: (m, 0)),
        out_shape=jax.ShapeDtypeStruct((n, out_f), jnp.float32),
        scratch_shapes=[pltpu.VMEM((n, out_f), jnp.float32)],
        compiler_params=pltpu.CompilerParams(
            dimension_semantics=("arbitrary",),
        ),
    )(vertex, weights, bias2, adj_distance, adj_angle)


# speedup vs baseline: 3.6645x; 1.0191x over previous
import jax
import jax.numpy as jnp
from jax.experimental import pallas as pl
from jax.experimental.pallas import tpu as pltpu


def _gc_kernel(v_ref, w_ref, b_ref, ad_ref, aa_ref, o_ref, s_ref):
    @pl.when(pl.program_id(0) == 0)
    def _():
        s_ref[...] = jnp.dot(v_ref[...], w_ref[...],
                             preferred_element_type=jnp.float32
                             ).astype(jnp.bfloat16)

    a = (ad_ref[...] + aa_ref[...]).astype(jnp.bfloat16)
    o_ref[...] = (jnp.dot(a, s_ref[...], preferred_element_type=jnp.float32)
                  + b_ref[...])


def kernel(vertex, adj_distance, adj_angle, weights, bias):
    n, in_f = vertex.shape
    out_f = weights.shape[1]
    bias2 = bias.reshape(1, out_f)

    tm = 80
    grid = (n // tm,)

    return pl.pallas_call(
        _gc_kernel,
        grid=grid,
        in_specs=[
            pl.BlockSpec((n, in_f), lambda m: (0, 0)),
            pl.BlockSpec((in_f, out_f), lambda m: (0, 0)),
            pl.BlockSpec((1, out_f), lambda m: (0, 0)),
            pl.BlockSpec((tm, n), lambda m: (m, 0)),
            pl.BlockSpec((tm, n), lambda m: (m, 0)),
        ],
        out_specs=pl.BlockSpec((tm, out_f), lambda m: (m, 0)),
        out_shape=jax.ShapeDtypeStruct((n, out_f), jnp.float32),
        scratch_shapes=[pltpu.VMEM((n, out_f), jnp.bfloat16)],
        compiler_params=pltpu.CompilerParams(
            dimension_semantics=("arbitrary",),
        ),
    )(vertex, weights, bias2, adj_distance, adj_angle)


# speedup vs baseline: 3.7804x; 1.0316x over previous
import jax
import jax.numpy as jnp
from jax.experimental import pallas as pl
from jax.experimental.pallas import tpu as pltpu


def _gc_kernel(v_ref, w_ref, b_ref, ad_ref, aa_ref, o_ref, s_ref):
    @pl.when(pl.program_id(0) == 0)
    def _():
        s_ref[...] = jnp.dot(v_ref[...], w_ref[...],
                             preferred_element_type=jnp.float32
                             ).astype(jnp.bfloat16)

    a = (ad_ref[...] + aa_ref[...]).astype(jnp.bfloat16)
    o_ref[...] = (jnp.dot(a, s_ref[...], preferred_element_type=jnp.float32)
                  + b_ref[...])


def kernel(vertex, adj_distance, adj_angle, weights, bias):
    n, in_f = vertex.shape
    out_f = weights.shape[1]
    bias2 = bias.reshape(1, out_f)

    tm = 200
    grid = (n // tm,)

    return pl.pallas_call(
        _gc_kernel,
        grid=grid,
        in_specs=[
            pl.BlockSpec((n, in_f), lambda m: (0, 0)),
            pl.BlockSpec((in_f, out_f), lambda m: (0, 0)),
            pl.BlockSpec((1, out_f), lambda m: (0, 0)),
            pl.BlockSpec((tm, n), lambda m: (m, 0)),
            pl.BlockSpec((tm, n), lambda m: (m, 0)),
        ],
        out_specs=pl.BlockSpec((tm, out_f), lambda m: (m, 0)),
        out_shape=jax.ShapeDtypeStruct((n, out_f), jnp.float32),
        scratch_shapes=[pltpu.VMEM((n, out_f), jnp.bfloat16)],
        compiler_params=pltpu.CompilerParams(
            dimension_semantics=("arbitrary",),
        ),
    )(vertex, weights, bias2, adj_distance, adj_angle)
